# Initial kernel scaffold; baseline (speedup 1.0000x reference)
import jax
import jax.numpy as jnp
from jax.experimental import pallas as pl


def kernel(x, linear_table, linear_bias, ffm_tables):
    raise NotImplementedError("write your pallas kernel here")


# speedup vs baseline: 37.1993x; 37.1993x over previous
import functools

import numpy as np
import jax
import jax.numpy as jnp
from jax import lax
from jax.experimental import pallas as pl
from jax.experimental.pallas import tpu as pltpu
from jax.experimental.pallas import tpu_sc as plsc

NUM_FIELDS = 26
VOCAB_PER_FIELD = 1000
VOCAB = NUM_FIELDS * VOCAB_PER_FIELD
EMBED_DIM = 16
BATCH = 4096
NUM_PAIRS = NUM_FIELDS * (NUM_FIELDS - 1) // 2

NC = 2
NS = 16
L = 16
NW = NC * NS
ROWS_PER_W = BATCH // NW
RV = ROWS_PER_W // L


@functools.partial(
    pl.kernel,
    out_type=jax.ShapeDtypeStruct((BATCH,), jnp.float32),
    mesh=plsc.VectorSubcoreMesh(core_axis_name="c", subcore_axis_name="s"),
    compiler_params=pltpu.CompilerParams(needs_layout_passes=False,
                                         use_tc_tiling_on_sc=False),
    scratch_types=[
        pltpu.VMEM((NUM_FIELDS * ROWS_PER_W,), jnp.int32),
        pltpu.VMEM((ROWS_PER_W,), jnp.int32),
        pltpu.VMEM((ROWS_PER_W,), jnp.int32),
        pltpu.VMEM((ROWS_PER_W,), jnp.int32),
        pltpu.VMEM((ROWS_PER_W,), jnp.int32),
        pltpu.VMEM((ROWS_PER_W,), jnp.int32),
        pltpu.VMEM((ROWS_PER_W,), jnp.int32),
        pltpu.VMEM((ROWS_PER_W, EMBED_DIM), jnp.float32),
        pltpu.VMEM((ROWS_PER_W, EMBED_DIM), jnp.float32),
        pltpu.VMEM((ROWS_PER_W, EMBED_DIM), jnp.float32),
        pltpu.VMEM((ROWS_PER_W, EMBED_DIM), jnp.float32),
        pltpu.VMEM((ROWS_PER_W, EMBED_DIM), jnp.float32),
        pltpu.VMEM((ROWS_PER_W, EMBED_DIM), jnp.float32),
        pltpu.VMEM((ROWS_PER_W, EMBED_DIM), jnp.float32),
        pltpu.VMEM((ROWS_PER_W,), jnp.float32),
        pltpu.VMEM((ROWS_PER_W,), jnp.float32),
        pltpu.VMEM((ROWS_PER_W,), jnp.float32),
        pltpu.VMEM((L,), jnp.float32),
        pltpu.SemaphoreType.DMA,
        pltpu.SemaphoreType.DMA,
        pltpu.SemaphoreType.DMA,
        pltpu.SemaphoreType.DMA,
    ],
)
def _ffm_sc(xT, ffm, lint, bias16, out,
            xflat, idx1a, idx2a, idx1b, idx2b, idx1c, idx2c,
            e1a, e2a, e1b, e2b, e1c, e2c,
            acc, accl, lbuf, ochunk, biasv, semA, semB, semC, semL):
    c = lax.axis_index("c")
    s = lax.axis_index("s")
    wid = s * NC + c
    base = wid * ROWS_PER_W

    for j in range(NUM_FIELDS):
        pltpu.sync_copy(xT.at[j, pl.ds(base, ROWS_PER_W)],
                        xflat.at[pl.ds(j * ROWS_PER_W, ROWS_PER_W)])
    pltpu.sync_copy(bias16, biasv)

    zero = jnp.zeros((L,), jnp.float32)
    for rr in range(ROWS_PER_W):
        acc[rr, :] = zero
    for v in range(RV):
        accl[pl.ds(v * L, L)] = zero

    iota = lax.iota(jnp.int32, L)

    def fire(i, j, idx1, idx2, e1, e2, sem):
        b1 = i * VOCAB + j * VOCAB_PER_FIELD
        b2 = j * VOCAB + i * VOCAB_PER_FIELD
        for v in range(RV):
            xj = plsc.load_gather(xflat, [j * ROWS_PER_W + v * L + iota])
            idx1[pl.ds(v * L, L)] = xj + b1
            xi = plsc.load_gather(xflat, [i * ROWS_PER_W + v * L + iota])
            idx2[pl.ds(v * L, L)] = xi + b2
        pltpu.async_copy(ffm.at[idx1], e1, sem)
        pltpu.async_copy(ffm.at[idx2], e2, sem)

    def drain_compute(idx1, idx2, e1, e2, sem):
        pltpu.make_async_copy(ffm.at[idx1], e1, sem).wait()
        pltpu.make_async_copy(ffm.at[idx2], e2, sem).wait()
        for rr in range(ROWS_PER_W):
            plsc.addupdate(acc.at[rr, :], e1[rr, :] * e2[rr, :])

    def advance(i, j):
        nj = j + 1
        wrap = nj >= NUM_FIELDS
        return lax.select(wrap, i + 1, i), lax.select(wrap, i + 2, nj)

    A = (idx1a, idx2a, e1a, e2a, semA)
    B = (idx1b, idx2b, e1b, e2b, semB)
    C = (idx1c, idx2c, e1c, e2c, semC)

    fire(jnp.int32(0), jnp.int32(1), *A)
    fire(jnp.int32(0), jnp.int32(2), *B)

    def pair_body(k, carry):
        i, j = carry
        fire(i, j, *C)
        i, j = advance(i, j)
        drain_compute(*A)
        fire(i, j, *A)
        i, j = advance(i, j)
        drain_compute(*B)
        fire(i, j, *B)
        i, j = advance(i, j)
        drain_compute(*C)
        return (i, j)

    i, j = lax.fori_loop(0, (NUM_PAIRS - 4) // 3, pair_body,
                         (jnp.int32(0), jnp.int32(3)))
    fire(i, j, *C)
    i, j = advance(i, j)
    drain_compute(*A)
    fire(i, j, *A)
    drain_compute(*B)
    drain_compute(*C)
    drain_compute(*A)

    def lin_body(j, carry):
        for v in range(RV):
            xj = plsc.load_gather(xflat, [j * ROWS_PER_W + v * L + iota])
            idx1a[pl.ds(v * L, L)] = xj + j * VOCAB_PER_FIELD
        pltpu.async_copy(lint.at[idx1a], lbuf, semL).wait()
        for v in range(RV):
            accl[pl.ds(v * L, L)] = (accl[pl.ds(v * L, L)]
                                     + lbuf[pl.ds(v * L, L)])
        return carry

    lax.fori_loop(0, NUM_FIELDS, lin_body, jnp.int32(0))

    bias_v = biasv[...]
    for g in range(RV):
        sacc = zero
        for dd in range(EMBED_DIM):
            col = plsc.load_gather(acc, [g * L + iota,
                                         jnp.full((L,), dd, jnp.int32)])
            sacc = sacc + col
        ochunk[pl.ds(g * L, L)] = sacc + accl[pl.ds(g * L, L)] + bias_v

    pltpu.sync_copy(ochunk, out.at[pl.ds(base, ROWS_PER_W)])


def kernel(x, linear_table, linear_bias, ffm_tables):
    xT = jnp.transpose(x)
    ffm_flat = ffm_tables.reshape(NUM_FIELDS * VOCAB, EMBED_DIM)
    lin_flat = linear_table.reshape(VOCAB)
    bias16 = jnp.broadcast_to(linear_bias, (L,)).astype(jnp.float32)
    return _ffm_sc(xT, ffm_flat, lin_flat, bias16)


# speedup vs baseline: 43.5382x; 1.1704x over previous
import functools

import numpy as np
import jax
import jax.numpy as jnp
from jax import lax
from jax.experimental import pallas as pl
from jax.experimental.pallas import tpu as pltpu
from jax.experimental.pallas import tpu_sc as plsc

NUM_FIELDS = 26
VOCAB_PER_FIELD = 1000
VOCAB = NUM_FIELDS * VOCAB_PER_FIELD
EMBED_DIM = 16
BATCH = 4096
NUM_PAIRS = NUM_FIELDS * (NUM_FIELDS - 1) // 2

NC = 2
NS = 16
L = 16
NW = NC * NS
ROWS_PER_W = BATCH // NW
RV = ROWS_PER_W // L


@functools.partial(
    pl.kernel,
    out_type=jax.ShapeDtypeStruct((BATCH,), jnp.float32),
    mesh=plsc.VectorSubcoreMesh(core_axis_name="c", subcore_axis_name="s"),
    compiler_params=pltpu.CompilerParams(needs_layout_passes=False,
                                         use_tc_tiling_on_sc=False),
    scratch_types=[
        pltpu.VMEM((NUM_FIELDS * ROWS_PER_W,), jnp.int32),
        pltpu.VMEM((ROWS_PER_W,), jnp.int32),
        pltpu.VMEM((ROWS_PER_W,), jnp.int32),
        pltpu.VMEM((ROWS_PER_W,), jnp.int32),
        pltpu.VMEM((ROWS_PER_W,), jnp.int32),
        pltpu.VMEM((ROWS_PER_W, EMBED_DIM), jnp.float32),
        pltpu.VMEM((ROWS_PER_W, EMBED_DIM), jnp.float32),
        pltpu.VMEM((ROWS_PER_W, EMBED_DIM), jnp.float32),
        pltpu.VMEM((ROWS_PER_W, EMBED_DIM), jnp.float32),
        pltpu.VMEM((ROWS_PER_W, EMBED_DIM), jnp.float32),
        pltpu.VMEM((ROWS_PER_W,), jnp.float32),
        pltpu.VMEM((ROWS_PER_W,), jnp.float32),
        pltpu.VMEM((ROWS_PER_W,), jnp.float32),
        pltpu.VMEM((L,), jnp.float32),
        pltpu.SemaphoreType.DMA,
        pltpu.SemaphoreType.DMA,
        pltpu.SemaphoreType.DMA,
    ],
)
def _ffm_sc(xT, ffm, lint, bias16, out,
            xflat, idx1a, idx2a, idx1b, idx2b, e1a, e2a, e1b, e2b,
            acc, accl, lbuf, ochunk, biasv, semA, semB, semL):
    c = lax.axis_index("c")
    s = lax.axis_index("s")
    wid = s * NC + c
    base = wid * ROWS_PER_W

    for j in range(NUM_FIELDS):
        pltpu.sync_copy(xT.at[j, pl.ds(base, ROWS_PER_W)],
                        xflat.at[pl.ds(j * ROWS_PER_W, ROWS_PER_W)])
    pltpu.sync_copy(bias16, biasv)

    zero = jnp.zeros((L,), jnp.float32)
    for rr in range(ROWS_PER_W):
        acc[rr, :] = zero
    for v in range(RV):
        accl[pl.ds(v * L, L)] = zero

    iota = lax.iota(jnp.int32, L)

    def fire(i, j, idx1, idx2, e1, e2, sem):
        b1 = i * VOCAB + j * VOCAB_PER_FIELD
        b2 = j * VOCAB + i * VOCAB_PER_FIELD
        for v in range(RV):
            xj = plsc.load_gather(xflat, [j * ROWS_PER_W + v * L + iota])
            idx1[pl.ds(v * L, L)] = xj + b1
            xi = plsc.load_gather(xflat, [i * ROWS_PER_W + v * L + iota])
            idx2[pl.ds(v * L, L)] = xi + b2
        pltpu.async_copy(ffm.at[idx1], e1, sem)
        pltpu.async_copy(ffm.at[idx2], e2, sem)

    def drain_compute(idx1, idx2, e1, e2, sem):
        pltpu.make_async_copy(ffm.at[idx1], e1, sem).wait()
        pltpu.make_async_copy(ffm.at[idx2], e2, sem).wait()
        for rr in range(ROWS_PER_W):
            plsc.addupdate(acc.at[rr, :], e1[rr, :] * e2[rr, :])

    def advance(i, j):
        nj = j + 1
        wrap = nj >= NUM_FIELDS
        return lax.select(wrap, i + 1, i), lax.select(wrap, i + 2, nj)

    fire(jnp.int32(0), jnp.int32(1), idx1a, idx2a, e1a, e2a, semA)

    def pair_body(k, carry):
        i, j = carry
        fire(i, j, idx1b, idx2b, e1b, e2b, semB)
        i, j = advance(i, j)
        drain_compute(idx1a, idx2a, e1a, e2a, semA)
        fire(i, j, idx1a, idx2a, e1a, e2a, semA)
        i, j = advance(i, j)
        drain_compute(idx1b, idx2b, e1b, e2b, semB)
        return (i, j)

    lax.fori_loop(0, (NUM_PAIRS - 1) // 2, pair_body,
                  (jnp.int32(0), jnp.int32(2)))
    drain_compute(idx1a, idx2a, e1a, e2a, semA)

    def lin_body(j, carry):
        for v in range(RV):
            xj = plsc.load_gather(xflat, [j * ROWS_PER_W + v * L + iota])
            idx1a[pl.ds(v * L, L)] = xj + j * VOCAB_PER_FIELD
        pltpu.async_copy(lint.at[idx1a], lbuf, semL).wait()
        for v in range(RV):
            accl[pl.ds(v * L, L)] = (accl[pl.ds(v * L, L)]
                                     + lbuf[pl.ds(v * L, L)])
        return carry

    lax.fori_loop(0, NUM_FIELDS, lin_body, jnp.int32(0))

    bias_v = biasv[...]
    for g in range(RV):
        sacc = zero
        for dd in range(EMBED_DIM):
            col = plsc.load_gather(acc, [g * L + iota,
                                         jnp.full((L,), dd, jnp.int32)])
            sacc = sacc + col
        ochunk[pl.ds(g * L, L)] = sacc + accl[pl.ds(g * L, L)] + bias_v

    pltpu.sync_copy(ochunk, out.at[pl.ds(base, ROWS_PER_W)])


def kernel(x, linear_table, linear_bias, ffm_tables):
    xT = jnp.transpose(x)
    ffm_flat = ffm_tables.reshape(NUM_FIELDS * VOCAB, EMBED_DIM)
    lin_flat = linear_table.reshape(VOCAB)
    bias16 = jnp.broadcast_to(linear_bias, (L,)).astype(jnp.float32)
    return _ffm_sc(xT, ffm_flat, lin_flat, bias16)
